# baseline (speedup 1.0000x reference)
import functools

import jax
import jax.numpy as jnp
from jax import lax
from jax.experimental import pallas as pl
from jax.experimental.pallas import tpu as pltpu
from jax.experimental.pallas import tpu_sc as plsc

_N_VOCAB = 1000
_D = 1000
_B = 1024
_T = 20
_NW = 32
_B_PER_W = _B // _NW
_EPC = 2
_ROWS = _EPC * _T
_N_CHUNKS = _B_PER_W // _EPC


@functools.partial(
    pl.kernel,
    mesh=plsc.VectorSubcoreMesh(core_axis_name="c", subcore_axis_name="s"),
    out_type=jax.ShapeDtypeStruct((_B, _T, _D), jnp.float32),
    compiler_params=pltpu.CompilerParams(use_tc_tiling_on_sc=False),
    scratch_types=[
        pltpu.VMEM((_B_PER_W * _T,), jnp.int32),
        pltpu.VMEM((2, _ROWS, _D), jnp.float32),
        pltpu.SemaphoreType.DMA,
        pltpu.SemaphoreType.DMA,
        pltpu.SemaphoreType.DMA,
        pltpu.SemaphoreType.DMA,
    ],
)
def _gather_rows(idx_hbm, table_hbm, out_hbm, idx_v, rows_v, gs0, gs1, ss0, ss1):
    wid = lax.axis_index("s") * 2 + lax.axis_index("c")
    wbase = wid * _B_PER_W
    pltpu.sync_copy(idx_hbm.at[wid], idx_v)
    gsem = [gs0, gs1]
    ssem = [ss0, ss1]
    gcp = [None, None]
    scp1 = [None, None]
    scp2 = [None, None]

    def gather(c, buf):
        return pltpu.async_copy(
            table_hbm.at[idx_v.at[pl.ds(c * _ROWS, _ROWS)]],
            rows_v.at[buf],
            gsem[buf],
        )

    gcp[0] = gather(0, 0)
    for c in range(_N_CHUNKS):
        b = c % 2
        nb = (c + 1) % 2
        if c + 1 < _N_CHUNKS:
            if scp1[nb] is not None:
                scp1[nb].wait()
                scp2[nb].wait()
            gcp[nb] = gather(c + 1, nb)
        gcp[b].wait()
        scp1[b] = pltpu.async_copy(
            rows_v.at[b, pl.ds(0, _T)], out_hbm.at[wbase + _EPC * c], ssem[b]
        )
        scp2[b] = pltpu.async_copy(
            rows_v.at[b, pl.ds(_T, _T)], out_hbm.at[wbase + _EPC * c + 1], ssem[b]
        )
    scp1[0].wait()
    scp2[0].wait()
    scp1[1].wait()
    scp2[1].wait()


def kernel(idx, table):
    idx_r = idx.reshape(_NW, _B_PER_W * _T)
    return _gather_rows(idx_r, table)


# speedup vs baseline: 1.0987x; 1.0987x over previous
import functools

import jax
import jax.numpy as jnp
from jax import lax
from jax.experimental import pallas as pl
from jax.experimental.pallas import tpu as pltpu
from jax.experimental.pallas import tpu_sc as plsc

_N_VOCAB = 1000
_D = 1000
_B = 1024
_T = 20
_NW = 32
_B_PER_W = _B // _NW
_EPC = 1
_ROWS = _EPC * _T
_N_CHUNKS = _B_PER_W // _EPC
_NSC = 16
_LOAD_ROWS = _N_VOCAB // _NSC
_LOAD_REM = _N_VOCAB - _NSC * _LOAD_ROWS


@functools.partial(
    pl.kernel,
    mesh=plsc.VectorSubcoreMesh(core_axis_name="c", subcore_axis_name="s"),
    out_type=jax.ShapeDtypeStruct((_B, _T, _D), jnp.float32),
    compiler_params=pltpu.CompilerParams(use_tc_tiling_on_sc=False),
    scratch_types=[
        pltpu.VMEM((_N_CHUNKS, _ROWS), jnp.int32),
        pltpu.VMEM((2, _ROWS, _D), jnp.float32),
        pltpu.VMEM_SHARED((_N_VOCAB, _D), jnp.float32),
        pltpu.SemaphoreType.DMA,
        pltpu.SemaphoreType.DMA,
        pltpu.SemaphoreType.DMA,
        pltpu.SemaphoreType.DMA,
    ],
)
def _gather_rows(idx_hbm, table_hbm, out_hbm, idx_v, rows_v, table_s, gs0, gs1, ss0, ss1):
    sid = lax.axis_index("s")
    wid = sid * 2 + lax.axis_index("c")
    wbase = wid * _B_PER_W
    pltpu.sync_copy(
        table_hbm.at[pl.ds(sid * _LOAD_ROWS, _LOAD_ROWS)],
        table_s.at[pl.ds(sid * _LOAD_ROWS, _LOAD_ROWS)],
    )

    @pl.when(sid == 0)
    def _():
        pltpu.sync_copy(
            table_hbm.at[pl.ds(_NSC * _LOAD_ROWS, _LOAD_REM)],
            table_s.at[pl.ds(_NSC * _LOAD_ROWS, _LOAD_REM)],
        )

    pltpu.sync_copy(idx_hbm.at[wid], idx_v)
    plsc.subcore_barrier()
    gsem = [gs0, gs1]
    ssem = [ss0, ss1]
    gcp = [None, None]
    scp = [None, None]

    def gather(c, buf):
        return pltpu.async_copy(
            table_s.at[idx_v.at[c]],
            rows_v.at[buf],
            gsem[buf],
        )

    gcp[0] = gather(0, 0)
    for c in range(_N_CHUNKS):
        b = c % 2
        nb = (c + 1) % 2
        if c + 1 < _N_CHUNKS:
            if scp[nb] is not None:
                scp[nb].wait()
            gcp[nb] = gather(c + 1, nb)
        gcp[b].wait()
        scp[b] = pltpu.async_copy(rows_v.at[b], out_hbm.at[wbase + c], ssem[b])
    scp[0].wait()
    scp[1].wait()


def kernel(idx, table):
    idx_r = idx.reshape(_NW, _N_CHUNKS, _ROWS)
    return _gather_rows(idx_r, table)


# speedup vs baseline: 1.1041x; 1.0050x over previous
import functools

import jax
import jax.numpy as jnp
from jax import lax
from jax.experimental import pallas as pl
from jax.experimental.pallas import tpu as pltpu
from jax.experimental.pallas import tpu_sc as plsc

_N_VOCAB = 1000
_D = 1000
_B = 1024
_T = 20
_NW = 32
_B_PER_W = _B // _NW
_EPC = 1
_ROWS = _EPC * _T
_N_CHUNKS = _B_PER_W // _EPC
_NSC = 16
_LOAD_ROWS = 64
_LOAD_REM = _N_VOCAB - (_NSC - 1) * _LOAD_ROWS


@functools.partial(
    pl.kernel,
    mesh=plsc.VectorSubcoreMesh(core_axis_name="c", subcore_axis_name="s"),
    out_type=jax.ShapeDtypeStruct((_B * _T, _D), jnp.float32),
    compiler_params=pltpu.CompilerParams(use_tc_tiling_on_sc=False),
    scratch_types=[
        pltpu.VMEM((_N_CHUNKS, _ROWS), jnp.int32),
        pltpu.VMEM((2, _ROWS, _D), jnp.float32),
        pltpu.VMEM_SHARED((_N_VOCAB, _D), jnp.float32),
        pltpu.SemaphoreType.DMA,
        pltpu.SemaphoreType.DMA,
        pltpu.SemaphoreType.DMA,
        pltpu.SemaphoreType.DMA,
    ],
)
def _gather_rows(idx_hbm, table_hbm, out_hbm, idx_v, rows_v, table_s, gs0, gs1, ss0, ss1):
    sid = lax.axis_index("s")
    wid = sid * 2 + lax.axis_index("c")
    wbase = wid * _B_PER_W
    @pl.when(sid < _NSC - 1)
    def _():
        pltpu.sync_copy(
            table_hbm.at[pl.ds(sid * _LOAD_ROWS, _LOAD_ROWS)],
            table_s.at[pl.ds(sid * _LOAD_ROWS, _LOAD_ROWS)],
        )

    @pl.when(sid == _NSC - 1)
    def _():
        pltpu.sync_copy(
            table_hbm.at[pl.ds((_NSC - 1) * _LOAD_ROWS, _LOAD_REM)],
            table_s.at[pl.ds((_NSC - 1) * _LOAD_ROWS, _LOAD_REM)],
        )

    pltpu.sync_copy(idx_hbm.at[wid], idx_v)
    plsc.subcore_barrier()
    gsem = [gs0, gs1]
    ssem = [ss0, ss1]
    gcp = [None, None]
    scp = [None, None]

    def gather(c, buf):
        return pltpu.async_copy(
            table_s.at[idx_v.at[c]],
            rows_v.at[buf],
            gsem[buf],
        )

    gcp[0] = gather(0, 0)
    for c in range(_N_CHUNKS):
        b = c % 2
        nb = (c + 1) % 2
        if c + 1 < _N_CHUNKS:
            if scp[nb] is not None:
                scp[nb].wait()
            gcp[nb] = gather(c + 1, nb)
        gcp[b].wait()
        scp[b] = pltpu.async_copy(
            rows_v.at[b], out_hbm.at[pl.ds((wbase + c) * _T, _T)], ssem[b]
        )
    scp[0].wait()
    scp[1].wait()


def kernel(idx, table):
    idx_r = idx.reshape(_NW, _N_CHUNKS, _ROWS)
    return _gather_rows(idx_r, table).reshape(_B, _T, _D)
